# baseline (speedup 1.0000x reference)
import functools

import jax
import jax.numpy as jnp
from jax import lax
from jax.experimental import pallas as pl
from jax.experimental.pallas import tpu as pltpu
from jax.experimental.pallas import tpu_sc as plsc

N = 10000
E = 320000
D = 128
H = 128
NC = 2
NS = 16
NW = NC * NS
EPW = E // NW
CW = 80
CH = EPW // CW
ASEG = 624
AROWS = N
DSEG = 640
NPAD = NS * DSEG

_mesh = plsc.VectorSubcoreMesh(core_axis_name="c", subcore_axis_name="s")


@functools.partial(
    pl.kernel,
    mesh=_mesh,
    out_type=jax.ShapeDtypeStruct((NC, NPAD), jnp.float32),
    scratch_types=[
        pltpu.VMEM((CH, CW), jnp.int32),
        pltpu.VMEM((EPW,), jnp.float32),
        pltpu.VMEM_SHARED((NPAD,), jnp.float32),
    ],
)
def _deg_call(dst_hbm, ew_hbm, zdeg_hbm, out_hbm, dst_v, ew_v, acc_sh):
    cid = lax.axis_index("c")
    sid = lax.axis_index("s")
    wid = cid * NS + sid
    pltpu.sync_copy(dst_hbm.at[wid], dst_v)
    pltpu.sync_copy(ew_hbm.at[wid], ew_v)
    pltpu.sync_copy(zdeg_hbm, acc_sh.at[pl.ds(sid * DSEG, DSEG)])
    plsc.subcore_barrier()

    def chunk(j, carry):
        pltpu.sync_copy(ew_v.at[pl.ds(j * CW, CW)],
                        acc_sh.at[dst_v.at[j]], add=True)
        return carry

    lax.fori_loop(0, CH, chunk, 0)
    plsc.subcore_barrier()
    pltpu.sync_copy(acc_sh.at[pl.ds(sid * DSEG, DSEG)],
                    out_hbm.at[cid, pl.ds(sid * DSEG, DSEG)])


@functools.partial(
    pl.kernel,
    mesh=_mesh,
    out_type=jax.ShapeDtypeStruct((NC, AROWS, H), jnp.float32),
    scratch_types=[
        pltpu.VMEM((EPW,), jnp.int32),
        pltpu.VMEM((EPW,), jnp.int32),
        pltpu.VMEM((1, CW), jnp.int32),
        pltpu.VMEM((EPW,), jnp.float32),
        pltpu.VMEM((CW, H), jnp.float32),
        pltpu.VMEM((CW, H), jnp.float32),
        pltpu.VMEM_SHARED((AROWS, H), jnp.float32),
        pltpu.SemaphoreType.DMA,
        pltpu.SemaphoreType.DMA,
    ],
)
def _agg_call(y_hbm, srcf_hbm, dstf_hbm, ew_hbm, out_hbm,
              src_v, dst_v, idxc_v, ew_v, rows_v, rows_w, acc_sh,
              sem_a, sem_b):
    cid = lax.axis_index("c")
    sid = lax.axis_index("s")
    wid = cid * NS + sid
    pltpu.sync_copy(srcf_hbm.at[wid], src_v)
    pltpu.sync_copy(dstf_hbm.at[wid], dst_v)
    pltpu.sync_copy(ew_hbm.at[wid], ew_v)

    def zrow(i, carry):
        for g in range(H // 16):
            rows_v[i, pl.ds(g * 16, 16)] = jnp.zeros((16,), jnp.float32)
        return carry

    lax.fori_loop(0, CW, zrow, 0)

    def zseg(i, carry):
        pltpu.sync_copy(rows_v, acc_sh.at[pl.ds(sid * ASEG + i * CW, CW)])
        return carry

    lax.fori_loop(0, ASEG // CW, zseg, 0)
    pltpu.sync_copy(
        rows_v.at[pl.ds(0, ASEG - (ASEG // CW) * CW)],
        acc_sh.at[pl.ds(sid * ASEG + (ASEG // CW) * CW,
                        ASEG - (ASEG // CW) * CW)])

    @pl.when(sid == NS - 1)
    def _():
        pltpu.sync_copy(rows_v.at[pl.ds(0, AROWS - NS * ASEG)],
                        acc_sh.at[pl.ds(NS * ASEG, AROWS - NS * ASEG)])

    plsc.subcore_barrier()

    NSPL = 5
    CWS = CW // NSPL

    def start_gather(j, buf, sem):
        jb = j * CW
        for q in range(NSPL):
            pltpu.async_copy(
                y_hbm.at[src_v.at[pl.ds(jb + q * CWS, CWS)]],
                buf.at[pl.ds(q * CWS, CWS)], sem)

    def wait_gather(j, buf, sem):
        jb = j * CW
        for q in range(NSPL):
            pltpu.make_async_copy(
                y_hbm.at[src_v.at[pl.ds(jb + q * CWS, CWS)]],
                buf.at[pl.ds(q * CWS, CWS)], sem).wait()

    def process(j, buf):
        jb = j * CW
        for k in range(CW // 16):
            ewv = ew_v[pl.ds(jb + k * 16, 16)]
            idxc_v[0, pl.ds(k * 16, 16)] = dst_v[pl.ds(jb + k * 16, 16)]
            for t in range(16):
                e = k * 16 + t
                s = jnp.broadcast_to(ewv[t], (16,))
                for g in range(H // 16):
                    buf[e, pl.ds(g * 16, 16)] = (
                        buf[e, pl.ds(g * 16, 16)] * s)
        pltpu.sync_copy(buf, acc_sh.at[idxc_v.at[0]], add=True)

    start_gather(0, rows_v, sem_a)

    def pair(j2, carry):
        j = 2 * j2
        wait_gather(j, rows_v, sem_a)
        start_gather(j + 1, rows_w, sem_b)
        process(j, rows_v)
        wait_gather(j + 1, rows_w, sem_b)
        start_gather(j + 2, rows_v, sem_a)
        process(j + 1, rows_w)
        return carry

    lax.fori_loop(0, (CH - 1) // 2, pair, 0)
    wait_gather(CH - 1, rows_v, sem_a)
    process(CH - 1, rows_v)
    plsc.subcore_barrier()
    pltpu.sync_copy(acc_sh.at[pl.ds(sid * ASEG, ASEG)],
                    out_hbm.at[cid, pl.ds(sid * ASEG, ASEG)])

    @pl.when(sid == NS - 1)
    def _():
        pltpu.sync_copy(
            acc_sh.at[pl.ds(NS * ASEG, AROWS - NS * ASEG)],
            out_hbm.at[cid, pl.ds(NS * ASEG, AROWS - NS * ASEG)])


_BR = 1000
_GRID = N // _BR


def _dis_of(degp_ref):
    deg = degp_ref[:, 0] + degp_ref[:, 1] + 1.0
    return lax.rsqrt(deg)


def _tc_b_body(degp_ref, x_ref, w1_ref, y1_ref):
    dis = _dis_of(degp_ref)
    xw = lax.dot_general(x_ref[...], w1_ref[...], (((1,), (1,)), ((), ())),
                         preferred_element_type=jnp.float32)
    y1_ref[...] = dis[:, None] * xw


def _tc_mid_body(degp_ref, p_ref, y_ref, b_ref, w_ref, fb_ref, s_ref,
                 yn_ref):
    dis = _dis_of(degp_ref)
    agg = p_ref[0] + p_ref[1] + y_ref[...]
    h = jnp.maximum(dis[:, None] * agg + b_ref[...], 0.0)
    xw = lax.dot_general(h, w_ref[...], (((1,), (1,)), ((), ())),
                         preferred_element_type=jnp.float32)
    s = s_ref[0, 0]
    scalef = dis[:, None] * s + (1.0 - s)
    yn_ref[...] = scalef * xw + fb_ref[...]


_DEG_SPEC = pl.BlockSpec((_BR, 2), lambda i: (i, 0))
_ROW_SPEC = pl.BlockSpec((_BR, H), lambda i: (i, 0))
_P_SPEC = pl.BlockSpec((2, _BR, H), lambda i: (0, i, 0))


def _tc_b(degp, x, W1):
    return pl.pallas_call(
        _tc_b_body,
        grid=(_GRID,),
        in_specs=[
            _DEG_SPEC,
            pl.BlockSpec((_BR, D), lambda i: (i, 0)),
            pl.BlockSpec((H, D), lambda i: (0, 0)),
        ],
        out_specs=_ROW_SPEC,
        out_shape=jax.ShapeDtypeStruct((N, H), jnp.float32),
    )(degp, x, W1)


def _tc_mid(degp, p, y, b, W, fb, s):
    return pl.pallas_call(
        _tc_mid_body,
        grid=(_GRID,),
        in_specs=[
            _DEG_SPEC,
            _P_SPEC,
            _ROW_SPEC,
            pl.BlockSpec((1, H), lambda i: (0, 0)),
            pl.BlockSpec((H, H), lambda i: (0, 0)),
            pl.BlockSpec((1, H), lambda i: (0, 0)),
            pl.BlockSpec((1, 1), lambda i: (0, 0)),
        ],
        out_specs=_ROW_SPEC,
        out_shape=jax.ShapeDtypeStruct((N, H), jnp.float32),
    )(degp, p, y, b, W, fb, s)


def kernel(x, edge_index, edge_weight, W1, b1, W2, b2, Wfc, bfc):
    srcf = edge_index[0].reshape(NW, EPW)
    dst = edge_index[1].reshape(NW, EPW)
    dst3 = edge_index[1].reshape(NW, CH, CW)
    ewr = edge_weight.reshape(NW, EPW)
    zdeg = jnp.zeros((DSEG,), jnp.float32)

    degp = _deg_call(dst3, ewr, zdeg).T
    y1 = _tc_b(degp, x, W1)

    b1r = b1.reshape(1, H)
    b2r = b2.reshape(1, H)
    wfc_p = jnp.zeros((H, H), jnp.float32).at[:3].set(Wfc)
    bfc_p = jnp.zeros((1, H), jnp.float32).at[0, :3].set(bfc)
    zrow1 = jnp.zeros((1, H), jnp.float32)
    trips = 2 + jnp.minimum(edge_index[0, 0], 0)

    def cond(carry):
        return carry[0] < trips

    def body(carry):
        l, y = carry
        first = l == 0
        b_l = jnp.where(first, b1r, b2r)
        w_l = jnp.where(first, W2, wfc_p)
        fb_l = jnp.where(first, zrow1, bfc_p)
        s_l = jnp.where(first, 1.0, 0.0).reshape(1, 1).astype(jnp.float32)
        p = _agg_call(y, srcf, dst, ewr)
        y_next = _tc_mid(degp, p, y, b_l, w_l, fb_l, s_l)
        return (l + 1, y_next)

    _, y_fin = lax.while_loop(cond, body, (jnp.int32(0), y1))
    return y_fin[:, :3]


# speedup vs baseline: 1.0002x; 1.0002x over previous
import functools

import jax
import jax.numpy as jnp
from jax import lax
from jax.experimental import pallas as pl
from jax.experimental.pallas import tpu as pltpu
from jax.experimental.pallas import tpu_sc as plsc

N = 10000
E = 320000
D = 128
H = 128
NC = 2
NS = 16
NW = NC * NS
EPW = E // NW
CW = 80
CH = EPW // CW
ASEG = 624
AROWS = N
DSEG = 640
NPAD = NS * DSEG

_mesh = plsc.VectorSubcoreMesh(core_axis_name="c", subcore_axis_name="s")


@functools.partial(
    pl.kernel,
    mesh=_mesh,
    out_type=jax.ShapeDtypeStruct((NC, NPAD), jnp.float32),
    scratch_types=[
        pltpu.VMEM((CH, CW), jnp.int32),
        pltpu.VMEM((EPW,), jnp.float32),
        pltpu.VMEM_SHARED((NPAD,), jnp.float32),
    ],
)
def _deg_call(dst_hbm, ew_hbm, zdeg_hbm, out_hbm, dst_v, ew_v, acc_sh):
    cid = lax.axis_index("c")
    sid = lax.axis_index("s")
    wid = cid * NS + sid
    pltpu.sync_copy(dst_hbm.at[wid], dst_v)
    pltpu.sync_copy(ew_hbm.at[wid], ew_v)
    pltpu.sync_copy(zdeg_hbm, acc_sh.at[pl.ds(sid * DSEG, DSEG)])
    plsc.subcore_barrier()

    def chunk(j, carry):
        pltpu.sync_copy(ew_v.at[pl.ds(j * CW, CW)],
                        acc_sh.at[dst_v.at[j]], add=True)
        return carry

    lax.fori_loop(0, CH, chunk, 0)
    plsc.subcore_barrier()
    pltpu.sync_copy(acc_sh.at[pl.ds(sid * DSEG, DSEG)],
                    out_hbm.at[cid, pl.ds(sid * DSEG, DSEG)])


@functools.partial(
    pl.kernel,
    mesh=_mesh,
    out_type=jax.ShapeDtypeStruct((NC, AROWS, H), jnp.float32),
    scratch_types=[
        pltpu.VMEM((EPW,), jnp.int32),
        pltpu.VMEM((EPW,), jnp.int32),
        pltpu.VMEM((1, CW), jnp.int32),
        pltpu.VMEM((EPW,), jnp.float32),
        pltpu.VMEM((CW, H), jnp.float32),
        pltpu.VMEM((CW, H), jnp.float32),
        pltpu.VMEM_SHARED((AROWS, H), jnp.float32),
        pltpu.SemaphoreType.DMA,
        pltpu.SemaphoreType.DMA,
    ],
)
def _agg_call(y_hbm, srcf_hbm, dstf_hbm, ew_hbm, out_hbm,
              src_v, dst_v, idxc_v, ew_v, rows_v, rows_w, acc_sh,
              sem_a, sem_b):
    cid = lax.axis_index("c")
    sid = lax.axis_index("s")
    wid = cid * NS + sid
    pltpu.sync_copy(srcf_hbm.at[wid], src_v)
    pltpu.sync_copy(dstf_hbm.at[wid], dst_v)
    pltpu.sync_copy(ew_hbm.at[wid], ew_v)

    def zrow(i, carry):
        for g in range(H // 16):
            rows_v[i, pl.ds(g * 16, 16)] = jnp.zeros((16,), jnp.float32)
        return carry

    lax.fori_loop(0, CW, zrow, 0)

    def zseg(i, carry):
        pltpu.sync_copy(rows_v, acc_sh.at[pl.ds(sid * ASEG + i * CW, CW)])
        return carry

    lax.fori_loop(0, ASEG // CW, zseg, 0)
    pltpu.sync_copy(
        rows_v.at[pl.ds(0, ASEG - (ASEG // CW) * CW)],
        acc_sh.at[pl.ds(sid * ASEG + (ASEG // CW) * CW,
                        ASEG - (ASEG // CW) * CW)])

    @pl.when(sid == NS - 1)
    def _():
        pltpu.sync_copy(rows_v.at[pl.ds(0, AROWS - NS * ASEG)],
                        acc_sh.at[pl.ds(NS * ASEG, AROWS - NS * ASEG)])

    plsc.subcore_barrier()

    NSPL = 2
    CWS = CW // NSPL

    def start_gather(j, buf, sem):
        jb = j * CW
        for q in range(NSPL):
            pltpu.async_copy(
                y_hbm.at[src_v.at[pl.ds(jb + q * CWS, CWS)]],
                buf.at[pl.ds(q * CWS, CWS)], sem)

    def wait_gather(j, buf, sem):
        jb = j * CW
        for q in range(NSPL):
            pltpu.make_async_copy(
                y_hbm.at[src_v.at[pl.ds(jb + q * CWS, CWS)]],
                buf.at[pl.ds(q * CWS, CWS)], sem).wait()

    def process(j, buf):
        jb = j * CW
        for k in range(CW // 16):
            ewv = ew_v[pl.ds(jb + k * 16, 16)]
            idxc_v[0, pl.ds(k * 16, 16)] = dst_v[pl.ds(jb + k * 16, 16)]
            for t in range(16):
                e = k * 16 + t
                s = jnp.broadcast_to(ewv[t], (16,))
                for g in range(H // 16):
                    buf[e, pl.ds(g * 16, 16)] = (
                        buf[e, pl.ds(g * 16, 16)] * s)
        pltpu.sync_copy(buf, acc_sh.at[idxc_v.at[0]], add=True)

    start_gather(0, rows_v, sem_a)

    def pair(j2, carry):
        j = 2 * j2
        wait_gather(j, rows_v, sem_a)
        start_gather(j + 1, rows_w, sem_b)
        process(j, rows_v)
        wait_gather(j + 1, rows_w, sem_b)
        start_gather(j + 2, rows_v, sem_a)
        process(j + 1, rows_w)
        return carry

    lax.fori_loop(0, (CH - 1) // 2, pair, 0)
    wait_gather(CH - 1, rows_v, sem_a)
    process(CH - 1, rows_v)
    plsc.subcore_barrier()
    pltpu.sync_copy(acc_sh.at[pl.ds(sid * ASEG, ASEG)],
                    out_hbm.at[cid, pl.ds(sid * ASEG, ASEG)])

    @pl.when(sid == NS - 1)
    def _():
        pltpu.sync_copy(
            acc_sh.at[pl.ds(NS * ASEG, AROWS - NS * ASEG)],
            out_hbm.at[cid, pl.ds(NS * ASEG, AROWS - NS * ASEG)])


_BR = 1000
_GRID = N // _BR


def _dis_of(degp_ref):
    deg = degp_ref[:, 0] + degp_ref[:, 1] + 1.0
    return lax.rsqrt(deg)


def _tc_b_body(degp_ref, x_ref, w1_ref, y1_ref):
    dis = _dis_of(degp_ref)
    xw = lax.dot_general(x_ref[...], w1_ref[...], (((1,), (1,)), ((), ())),
                         preferred_element_type=jnp.float32)
    y1_ref[...] = dis[:, None] * xw


def _tc_mid_body(degp_ref, p_ref, y_ref, b_ref, w_ref, fb_ref, s_ref,
                 yn_ref):
    dis = _dis_of(degp_ref)
    agg = p_ref[0] + p_ref[1] + y_ref[...]
    h = jnp.maximum(dis[:, None] * agg + b_ref[...], 0.0)
    xw = lax.dot_general(h, w_ref[...], (((1,), (1,)), ((), ())),
                         preferred_element_type=jnp.float32)
    s = s_ref[0, 0]
    scalef = dis[:, None] * s + (1.0 - s)
    yn_ref[...] = scalef * xw + fb_ref[...]


_DEG_SPEC = pl.BlockSpec((_BR, 2), lambda i: (i, 0))
_ROW_SPEC = pl.BlockSpec((_BR, H), lambda i: (i, 0))
_P_SPEC = pl.BlockSpec((2, _BR, H), lambda i: (0, i, 0))


def _tc_b(degp, x, W1):
    return pl.pallas_call(
        _tc_b_body,
        grid=(_GRID,),
        in_specs=[
            _DEG_SPEC,
            pl.BlockSpec((_BR, D), lambda i: (i, 0)),
            pl.BlockSpec((H, D), lambda i: (0, 0)),
        ],
        out_specs=_ROW_SPEC,
        out_shape=jax.ShapeDtypeStruct((N, H), jnp.float32),
    )(degp, x, W1)


def _tc_mid(degp, p, y, b, W, fb, s):
    return pl.pallas_call(
        _tc_mid_body,
        grid=(_GRID,),
        in_specs=[
            _DEG_SPEC,
            _P_SPEC,
            _ROW_SPEC,
            pl.BlockSpec((1, H), lambda i: (0, 0)),
            pl.BlockSpec((H, H), lambda i: (0, 0)),
            pl.BlockSpec((1, H), lambda i: (0, 0)),
            pl.BlockSpec((1, 1), lambda i: (0, 0)),
        ],
        out_specs=_ROW_SPEC,
        out_shape=jax.ShapeDtypeStruct((N, H), jnp.float32),
    )(degp, p, y, b, W, fb, s)


def kernel(x, edge_index, edge_weight, W1, b1, W2, b2, Wfc, bfc):
    srcf = edge_index[0].reshape(NW, EPW)
    dst = edge_index[1].reshape(NW, EPW)
    dst3 = edge_index[1].reshape(NW, CH, CW)
    ewr = edge_weight.reshape(NW, EPW)
    zdeg = jnp.zeros((DSEG,), jnp.float32)

    degp = _deg_call(dst3, ewr, zdeg).T
    y1 = _tc_b(degp, x, W1)

    b1r = b1.reshape(1, H)
    b2r = b2.reshape(1, H)
    wfc_p = jnp.zeros((H, H), jnp.float32).at[:3].set(Wfc)
    bfc_p = jnp.zeros((1, H), jnp.float32).at[0, :3].set(bfc)
    zrow1 = jnp.zeros((1, H), jnp.float32)
    trips = 2 + jnp.minimum(edge_index[0, 0], 0)

    def cond(carry):
        return carry[0] < trips

    def body(carry):
        l, y = carry
        first = l == 0
        b_l = jnp.where(first, b1r, b2r)
        w_l = jnp.where(first, W2, wfc_p)
        fb_l = jnp.where(first, zrow1, bfc_p)
        s_l = jnp.where(first, 1.0, 0.0).reshape(1, 1).astype(jnp.float32)
        p = _agg_call(y, srcf, dst, ewr)
        y_next = _tc_mid(degp, p, y, b_l, w_l, fb_l, s_l)
        return (l + 1, y_next)

    _, y_fin = lax.while_loop(cond, body, (jnp.int32(0), y1))
    return y_fin[:, :3]


# speedup vs baseline: 1.0372x; 1.0370x over previous
import functools

import jax
import jax.numpy as jnp
from jax import lax
from jax.experimental import pallas as pl
from jax.experimental.pallas import tpu as pltpu
from jax.experimental.pallas import tpu_sc as plsc

N = 10000
E = 320000
D = 128
H = 128
NC = 2
NS = 16
NW = NC * NS
EPW = E // NW
CW = 80
CH = EPW // CW
ASEG = 624
AROWS = N
DSEG = 640
NPAD = NS * DSEG

_mesh = plsc.VectorSubcoreMesh(core_axis_name="c", subcore_axis_name="s")


@functools.partial(
    pl.kernel,
    mesh=_mesh,
    out_type=jax.ShapeDtypeStruct((NC, NPAD), jnp.float32),
    scratch_types=[
        pltpu.VMEM((CH, CW), jnp.int32),
        pltpu.VMEM((EPW,), jnp.float32),
        pltpu.VMEM_SHARED((NPAD,), jnp.float32),
    ],
)
def _deg_call(dst_hbm, ew_hbm, zdeg_hbm, out_hbm, dst_v, ew_v, acc_sh):
    cid = lax.axis_index("c")
    sid = lax.axis_index("s")
    wid = cid * NS + sid
    pltpu.sync_copy(dst_hbm.at[wid], dst_v)
    pltpu.sync_copy(ew_hbm.at[wid], ew_v)
    pltpu.sync_copy(zdeg_hbm, acc_sh.at[pl.ds(sid * DSEG, DSEG)])
    plsc.subcore_barrier()

    def chunk(j, carry):
        pltpu.sync_copy(ew_v.at[pl.ds(j * CW, CW)],
                        acc_sh.at[dst_v.at[j]], add=True)
        return carry

    lax.fori_loop(0, CH, chunk, 0)
    plsc.subcore_barrier()
    pltpu.sync_copy(acc_sh.at[pl.ds(sid * DSEG, DSEG)],
                    out_hbm.at[cid, pl.ds(sid * DSEG, DSEG)])


@functools.partial(
    pl.kernel,
    mesh=_mesh,
    out_type=jax.ShapeDtypeStruct((NC, AROWS, H), jnp.float32),
    scratch_types=[
        pltpu.VMEM((EPW,), jnp.int32),
        pltpu.VMEM((EPW,), jnp.int32),
        pltpu.VMEM((1, CW), jnp.int32),
        pltpu.VMEM((EPW,), jnp.float32),
        pltpu.VMEM((CW, H), jnp.float32),
        pltpu.VMEM((CW, H), jnp.float32),
        pltpu.VMEM_SHARED((AROWS, H), jnp.float32),
        pltpu.SemaphoreType.DMA,
        pltpu.SemaphoreType.DMA,
        pltpu.SemaphoreType.DMA,
        pltpu.SemaphoreType.DMA,
    ],
)
def _agg_call(y_hbm, srcf_hbm, dstf_hbm, ew_hbm, out_hbm,
              src_v, dst_v, idxc_v, ew_v, rows_v, rows_w, acc_sh,
              sem_a, sem_a2, sem_b, sem_b2):
    cid = lax.axis_index("c")
    sid = lax.axis_index("s")
    wid = cid * NS + sid
    pltpu.sync_copy(srcf_hbm.at[wid], src_v)
    pltpu.sync_copy(dstf_hbm.at[wid], dst_v)
    pltpu.sync_copy(ew_hbm.at[wid], ew_v)

    def zrow(i, carry):
        for g in range(H // 16):
            rows_v[i, pl.ds(g * 16, 16)] = jnp.zeros((16,), jnp.float32)
        return carry

    lax.fori_loop(0, CW, zrow, 0)

    def zseg(i, carry):
        pltpu.sync_copy(rows_v, acc_sh.at[pl.ds(sid * ASEG + i * CW, CW)])
        return carry

    lax.fori_loop(0, ASEG // CW, zseg, 0)
    pltpu.sync_copy(
        rows_v.at[pl.ds(0, ASEG - (ASEG // CW) * CW)],
        acc_sh.at[pl.ds(sid * ASEG + (ASEG // CW) * CW,
                        ASEG - (ASEG // CW) * CW)])

    @pl.when(sid == NS - 1)
    def _():
        pltpu.sync_copy(rows_v.at[pl.ds(0, AROWS - NS * ASEG)],
                        acc_sh.at[pl.ds(NS * ASEG, AROWS - NS * ASEG)])

    plsc.subcore_barrier()

    CW2 = CW // 2

    def start_gather(j, buf, sem, sem2):
        jb = j * CW
        pltpu.async_copy(y_hbm.at[src_v.at[pl.ds(jb, CW2)]],
                         buf.at[pl.ds(0, CW2)], sem)
        pltpu.async_copy(y_hbm.at[src_v.at[pl.ds(jb + CW2, CW2)]],
                         buf.at[pl.ds(CW2, CW2)], sem2)

    def wait_gather(j, buf, sem, sem2):
        jb = j * CW
        pltpu.make_async_copy(y_hbm.at[src_v.at[pl.ds(jb, CW2)]],
                              buf.at[pl.ds(0, CW2)], sem).wait()
        pltpu.make_async_copy(y_hbm.at[src_v.at[pl.ds(jb + CW2, CW2)]],
                              buf.at[pl.ds(CW2, CW2)], sem2).wait()

    def process(j, buf):
        jb = j * CW
        for k in range(CW // 16):
            ewv = ew_v[pl.ds(jb + k * 16, 16)]
            idxc_v[0, pl.ds(k * 16, 16)] = dst_v[pl.ds(jb + k * 16, 16)]
            for t in range(16):
                e = k * 16 + t
                s = jnp.broadcast_to(ewv[t], (16,))
                for g in range(H // 16):
                    buf[e, pl.ds(g * 16, 16)] = (
                        buf[e, pl.ds(g * 16, 16)] * s)
        pltpu.sync_copy(buf, acc_sh.at[idxc_v.at[0]], add=True)

    start_gather(0, rows_v, sem_a, sem_a2)

    def pair(j2, carry):
        j = 2 * j2
        wait_gather(j, rows_v, sem_a, sem_a2)
        start_gather(j + 1, rows_w, sem_b, sem_b2)
        process(j, rows_v)
        wait_gather(j + 1, rows_w, sem_b, sem_b2)
        start_gather(j + 2, rows_v, sem_a, sem_a2)
        process(j + 1, rows_w)
        return carry

    lax.fori_loop(0, (CH - 1) // 2, pair, 0)
    wait_gather(CH - 1, rows_v, sem_a, sem_a2)
    process(CH - 1, rows_v)
    plsc.subcore_barrier()
    pltpu.sync_copy(acc_sh.at[pl.ds(sid * ASEG, ASEG)],
                    out_hbm.at[cid, pl.ds(sid * ASEG, ASEG)])

    @pl.when(sid == NS - 1)
    def _():
        pltpu.sync_copy(
            acc_sh.at[pl.ds(NS * ASEG, AROWS - NS * ASEG)],
            out_hbm.at[cid, pl.ds(NS * ASEG, AROWS - NS * ASEG)])


_BR = 1000
_GRID = N // _BR


def _dis_of(degp_ref):
    deg = degp_ref[:, 0] + degp_ref[:, 1] + 1.0
    return lax.rsqrt(deg)


def _tc_b_body(degp_ref, x_ref, w1_ref, y1_ref):
    dis = _dis_of(degp_ref)
    xw = lax.dot_general(x_ref[...], w1_ref[...], (((1,), (1,)), ((), ())),
                         preferred_element_type=jnp.float32)
    y1_ref[...] = dis[:, None] * xw


def _tc_mid_body(degp_ref, p_ref, y_ref, b_ref, w_ref, h_ref, yn_ref):
    dis = _dis_of(degp_ref)
    agg = p_ref[0] + p_ref[1] + y_ref[...]
    h = jnp.maximum(dis[:, None] * agg + b_ref[...], 0.0)
    h_ref[...] = h
    xw = lax.dot_general(h, w_ref[...], (((1,), (1,)), ((), ())),
                         preferred_element_type=jnp.float32)
    yn_ref[...] = dis[:, None] * xw


def _tc_f_body(h_ref, wfc_ref, bfc_ref, o_ref):
    o_ref[...] = lax.dot_general(
        h_ref[...], wfc_ref[...], (((1,), (1,)), ((), ())),
        preferred_element_type=jnp.float32) + bfc_ref[...]


_DEG_SPEC = pl.BlockSpec((_BR, 2), lambda i: (i, 0))
_ROW_SPEC = pl.BlockSpec((_BR, H), lambda i: (i, 0))
_P_SPEC = pl.BlockSpec((2, _BR, H), lambda i: (0, i, 0))


def _tc_b(degp, x, W1):
    return pl.pallas_call(
        _tc_b_body,
        grid=(_GRID,),
        in_specs=[
            _DEG_SPEC,
            pl.BlockSpec((_BR, D), lambda i: (i, 0)),
            pl.BlockSpec((H, D), lambda i: (0, 0)),
        ],
        out_specs=_ROW_SPEC,
        out_shape=jax.ShapeDtypeStruct((N, H), jnp.float32),
    )(degp, x, W1)


def _tc_mid(degp, p, y, b, W):
    return pl.pallas_call(
        _tc_mid_body,
        grid=(_GRID,),
        in_specs=[
            _DEG_SPEC,
            _P_SPEC,
            _ROW_SPEC,
            pl.BlockSpec((1, H), lambda i: (0, 0)),
            pl.BlockSpec((H, H), lambda i: (0, 0)),
        ],
        out_specs=[_ROW_SPEC, _ROW_SPEC],
        out_shape=[jax.ShapeDtypeStruct((N, H), jnp.float32),
                   jax.ShapeDtypeStruct((N, H), jnp.float32)],
    )(degp, p, y, b, W)


def _tc_f(h, wfc_p, bfc_p):
    return pl.pallas_call(
        _tc_f_body,
        grid=(_GRID,),
        in_specs=[
            _ROW_SPEC,
            pl.BlockSpec((8, H), lambda i: (0, 0)),
            pl.BlockSpec((1, 8), lambda i: (0, 0)),
        ],
        out_specs=pl.BlockSpec((_BR, 8), lambda i: (i, 0)),
        out_shape=jax.ShapeDtypeStruct((N, 8), jnp.float32),
    )(h, wfc_p, bfc_p)


def kernel(x, edge_index, edge_weight, W1, b1, W2, b2, Wfc, bfc):
    srcf = edge_index[0].reshape(NW, EPW)
    dst = edge_index[1].reshape(NW, EPW)
    dst3 = edge_index[1].reshape(NW, CH, CW)
    ewr = edge_weight.reshape(NW, EPW)
    zdeg = jnp.zeros((DSEG,), jnp.float32)

    degp = _deg_call(dst3, ewr, zdeg).T
    y1 = _tc_b(degp, x, W1)

    b1r = b1.reshape(1, H)
    b2r = b2.reshape(1, H)
    trips = 2 + jnp.minimum(edge_index[0, 0], 0)

    def cond(carry):
        return carry[0] < trips

    def body(carry):
        l, y, _ = carry
        b_l = jnp.where(l == 0, b1r, b2r)
        p = _agg_call(y, srcf, dst, ewr)
        h, y_next = _tc_mid(degp, p, y, b_l, W2)
        return (l + 1, y_next, h)

    _, _, h_fin = lax.while_loop(
        cond, body, (jnp.int32(0), y1, jnp.zeros_like(y1)))

    wfc_p = jnp.zeros((8, H), jnp.float32).at[:3].set(Wfc)
    bfc_p = jnp.zeros((1, 8), jnp.float32).at[0, :3].set(bfc)
    out8 = _tc_f(h_fin, wfc_p, bfc_p)
    return out8[:, :3]
